# baseline (speedup 1.0000x reference)
import functools

import jax
import jax.numpy as jnp
from jax import lax
from jax.experimental import pallas as pl
from jax.experimental.pallas import tpu as pltpu
from jax.experimental.pallas import tpu_sc as plsc

VOCAB = 1000000
DIM = 64
B = 16384
L = 20
W = 2 * DIM

NC = 2
NS = 16
NW = NC * NS

NTOT = B * L
PER_W = NTOT // NW
SUB = 128
NSUB = 2
CH = SUB * NSUB
NCHUNK = PER_W // CH


def _sc_kernel(kc_hbm, kx_hbm, oc_hbm, ox_hbm, in_hbm, out_emb_hbm, out_hbm,
               idx_c, idx_x, off_c, off_x, crows, xrows, scores, sem):
    wid = lax.axis_index("s") * NC + lax.axis_index("c")

    def chunk_body(c, _):
        pltpu.sync_copy(kc_hbm.at[wid, c], idx_c)
        pltpu.sync_copy(kx_hbm.at[wid, c], idx_x)
        pltpu.sync_copy(oc_hbm.at[wid, c], off_c)
        pltpu.sync_copy(ox_hbm.at[wid, c], off_x)

        copies = []
        for j in range(NSUB):
            copies.append(
                pltpu.async_copy(in_hbm.at[idx_c.at[j]], crows.at[j], sem))
            copies.append(
                pltpu.async_copy(out_emb_hbm.at[idx_x.at[j]], xrows.at[j], sem))
        for cp in copies:
            cp.wait()

        iota16 = lax.iota(jnp.int32, 16)
        for j in range(NSUB):
            def grp_body(g, _):
                out16 = jnp.zeros((16,), jnp.float32)
                ocv = off_c[j, pl.ds(g * 16, 16)]
                oxv = off_x[j, pl.ds(g * 16, 16)]
                for p in range(16):
                    i = g * 16 + p
                    poc = ocv[p]
                    pox = oxv[p]
                    acc = (crows[j, i, pl.ds(poc, 16)]
                           * xrows[j, i, pl.ds(pox, 16)]
                           + crows[j, i, pl.ds(poc + 16, 16)]
                           * xrows[j, i, pl.ds(pox + 16, 16)])
                    acc = acc + (crows[j, i, pl.ds(poc + 32, 16)]
                                 * xrows[j, i, pl.ds(pox + 32, 16)])
                    acc = acc + (crows[j, i, pl.ds(poc + 48, 16)]
                                 * xrows[j, i, pl.ds(pox + 48, 16)])
                    s = jnp.sum(acc)
                    out16 = jnp.where(iota16 == p, lax.broadcast(s, (16,)), out16)
                scores[j, pl.ds(g * 16, 16)] = out16
                return 0
            lax.fori_loop(0, SUB // 16, grp_body, 0)

        pltpu.sync_copy(scores, out_hbm.at[wid, c])
        return 0

    lax.fori_loop(0, NCHUNK, chunk_body, 0)


@jax.jit
def _run(kc, kx, oc, ox, in_emb2, out_emb2):
    mesh = plsc.VectorSubcoreMesh(core_axis_name="c", subcore_axis_name="s",
                                  num_cores=NC, num_subcores=NS)
    kfn = pl.kernel(
        _sc_kernel,
        out_type=jax.ShapeDtypeStruct((NW, NCHUNK, NSUB, SUB), jnp.float32),
        mesh=mesh,
        compiler_params=pltpu.CompilerParams(needs_layout_passes=False,
                                             use_tc_tiling_on_sc=True),
        scratch_types=[
            pltpu.VMEM((NSUB, SUB), jnp.int32),
            pltpu.VMEM((NSUB, SUB), jnp.int32),
            pltpu.VMEM((NSUB, SUB), jnp.int32),
            pltpu.VMEM((NSUB, SUB), jnp.int32),
            pltpu.VMEM((NSUB, SUB, W), jnp.float32),
            pltpu.VMEM((NSUB, SUB, W), jnp.float32),
            pltpu.VMEM((NSUB, SUB), jnp.float32),
            pltpu.SemaphoreType.DMA,
        ],
    )
    return kfn(kc, kx, oc, ox, in_emb2, out_emb2)


def kernel(center_words, context_words, in_embeddings, out_embeddings):
    cw = center_words.reshape(NW, NCHUNK, NSUB, SUB).astype(jnp.int32)
    xw = context_words.reshape(NW, NCHUNK, NSUB, SUB).astype(jnp.int32)
    kc = cw >> 1
    kx = xw >> 1
    oc = (cw & 1) * DIM
    ox = (xw & 1) * DIM
    in2 = in_embeddings.reshape(VOCAB // 2, W)
    out2 = out_embeddings.reshape(VOCAB // 2, W)
    scores = _run(kc, kx, oc, ox, in2, out2)
    return scores.reshape(B, L)


# speedup vs baseline: 1.1970x; 1.1970x over previous
import functools

import jax
import jax.numpy as jnp
from jax import lax
from jax.experimental import pallas as pl
from jax.experimental.pallas import tpu as pltpu
from jax.experimental.pallas import tpu_sc as plsc

VOCAB = 1000000
DIM = 64
B = 16384
L = 20
W = 128

NC = 2
NS = 16
NW = NC * NS

NTOT = B * L
PER_W = NTOT // NW
SUB = 128
NSUB = 2
CH = SUB * NSUB
NCHUNK = PER_W // CH


def _sc_kernel(cw_hbm, xw_hbm, in_hbm, out_emb_hbm, out_hbm,
               idx_c, idx_x, crows, xrows, scores, sem):
    wid = lax.axis_index("s") * NC + lax.axis_index("c")

    def chunk_body(c, _):
        pltpu.sync_copy(cw_hbm.at[wid, c], idx_c)
        pltpu.sync_copy(xw_hbm.at[wid, c], idx_x)

        copies = []
        for j in range(NSUB):
            copies.append(
                pltpu.async_copy(in_hbm.at[idx_c.at[j]], crows.at[j], sem))
            copies.append(
                pltpu.async_copy(out_emb_hbm.at[idx_x.at[j]], xrows.at[j], sem))
        for cp in copies:
            cp.wait()

        iota16 = lax.iota(jnp.int32, 16)
        for j in range(NSUB):
            def grp_body(g, _):
                out16 = jnp.zeros((16,), jnp.float32)
                for p in range(16):
                    i = g * 16 + p
                    acc = (crows[j, i, pl.ds(0, 16)] * xrows[j, i, pl.ds(0, 16)]
                           + crows[j, i, pl.ds(16, 16)] * xrows[j, i, pl.ds(16, 16)])
                    acc = acc + crows[j, i, pl.ds(32, 16)] * xrows[j, i, pl.ds(32, 16)]
                    acc = acc + crows[j, i, pl.ds(48, 16)] * xrows[j, i, pl.ds(48, 16)]
                    s = jnp.sum(acc)
                    out16 = jnp.where(iota16 == p, lax.broadcast(s, (16,)), out16)
                scores[j, pl.ds(g * 16, 16)] = out16
                return 0
            lax.fori_loop(0, SUB // 16, grp_body, 0)

        pltpu.sync_copy(scores, out_hbm.at[wid, c])
        return 0

    lax.fori_loop(0, NCHUNK, chunk_body, 0)


@jax.jit
def _run(cw, xw, in_pad, out_pad):
    mesh = plsc.VectorSubcoreMesh(core_axis_name="c", subcore_axis_name="s",
                                  num_cores=NC, num_subcores=NS)
    kfn = pl.kernel(
        _sc_kernel,
        out_type=jax.ShapeDtypeStruct((NW, NCHUNK, NSUB, SUB), jnp.float32),
        mesh=mesh,
        compiler_params=pltpu.CompilerParams(needs_layout_passes=False,
                                             use_tc_tiling_on_sc=False),
        scratch_types=[
            pltpu.VMEM((NSUB, SUB), jnp.int32),
            pltpu.VMEM((NSUB, SUB), jnp.int32),
            pltpu.VMEM((NSUB, SUB, W), jnp.float32),
            pltpu.VMEM((NSUB, SUB, W), jnp.float32),
            pltpu.VMEM((NSUB, SUB), jnp.float32),
            pltpu.SemaphoreType.DMA,
        ],
    )
    return kfn(cw, xw, in_pad, out_pad)


def kernel(center_words, context_words, in_embeddings, out_embeddings):
    cw = center_words.T.reshape(NW, NCHUNK, NSUB, SUB).astype(jnp.int32)
    xw = context_words.T.reshape(NW, NCHUNK, NSUB, SUB).astype(jnp.int32)
    in_pad = jnp.pad(in_embeddings, ((0, 0), (0, W - DIM)))
    out_pad = jnp.pad(out_embeddings, ((0, 0), (0, W - DIM)))
    scores = _run(cw, xw, in_pad, out_pad)
    return scores.reshape(L, B).T
